# Initial kernel scaffold; baseline (speedup 1.0000x reference)
import functools

import jax
import jax.numpy as jnp
from jax import lax
from jax.experimental import pallas as pl
from jax.experimental.pallas import tpu as pltpu
from jax.experimental.pallas import tpu_sc as plsc

N = 10000
D = 256
HID = 256
OUT = 64
MAXD = 17
G = 8

NC = 2
NS = 16
NW = NC * NS
PW = 320
NPAD = NW * PW
CHUNK = 80
NCH = PW // CHUNK

RBLK = 1024
NB = NPAD // RBLK


def _sc_gather_sum(table, idx4):
    mesh = plsc.VectorSubcoreMesh(core_axis_name="c", subcore_axis_name="s")

    @functools.partial(
        pl.kernel,
        out_type=jax.ShapeDtypeStruct((NPAD, D), jnp.float32),
        mesh=mesh,
        scratch_types=[
            pltpu.VMEM((MAXD, CHUNK), jnp.int32),
            pltpu.VMEM((CHUNK, D), jnp.float32),
            pltpu.SemaphoreType.DMA,
        ],
    )
    def k(table_hbm, idx_hbm, out_hbm, idx_v, acc_v, sem):
        wid = lax.axis_index("s") * NC + lax.axis_index("c")
        base = wid * PW
        for c in range(NCH):
            cb = base + c * CHUNK
            pltpu.sync_copy(idx_hbm.at[wid, c], idx_v)
            pltpu.async_copy(table_hbm.at[idx_v.at[0]], acc_v, sem).wait()
            cps = [
                pltpu.async_copy(table_hbm.at[idx_v.at[j]], acc_v, sem, add=True)
                for j in range(1, MAXD)
            ]
            for cp in cps:
                cp.wait()
            pltpu.sync_copy(acc_v, out_hbm.at[pl.ds(cb, CHUNK)])

    return k(table, idx4)


def _tc_layer1(agg1, x_pad, segT, W0, b0):
    def body(agg_ref, x_ref, segT_ref, w_ref, b_ref, h_ref, ge0_ref, ge1_ref):
        i = pl.program_id(0)
        h = jnp.dot(agg_ref[...], w_ref[...], preferred_element_type=jnp.float32)
        h = jnp.maximum(h + b_ref[...], 0.0)
        h_ref[...] = h
        s = segT_ref[...]
        p0 = jnp.dot(s, x_ref[...], preferred_element_type=jnp.float32)
        p1 = jnp.dot(s, h, preferred_element_type=jnp.float32)

        @pl.when(i == 0)
        def _():
            ge0_ref[...] = p0
            ge1_ref[...] = p1

        @pl.when(i > 0)
        def _():
            ge0_ref[...] += p0
            ge1_ref[...] += p1

    return pl.pallas_call(
        body,
        grid=(NB,),
        in_specs=[
            pl.BlockSpec((RBLK, D), lambda i: (i, 0)),
            pl.BlockSpec((RBLK, D), lambda i: (i, 0)),
            pl.BlockSpec((G, RBLK), lambda i: (0, i)),
            pl.BlockSpec((D, HID), lambda i: (0, 0)),
            pl.BlockSpec((1, HID), lambda i: (0, 0)),
        ],
        out_specs=[
            pl.BlockSpec((RBLK, HID), lambda i: (i, 0)),
            pl.BlockSpec((G, D), lambda i: (0, 0)),
            pl.BlockSpec((G, HID), lambda i: (0, 0)),
        ],
        out_shape=[
            jax.ShapeDtypeStruct((NPAD, HID), jnp.float32),
            jax.ShapeDtypeStruct((G, D), jnp.float32),
            jax.ShapeDtypeStruct((G, HID), jnp.float32),
        ],
    )(agg1, x_pad, segT, W0, b0)


def _tc_layer2(agg2, segT, W1, b1, ge0, ge1, C0w, C1w, C2w, cb):
    def body(agg_ref, segT_ref, w_ref, b_ref, ge0_ref, ge1_ref,
             c0_ref, c1_ref, c2_ref, cb_ref, preds_ref, acc_ref):
        i = pl.program_id(0)
        h = jnp.dot(agg_ref[...], w_ref[...], preferred_element_type=jnp.float32)
        h = jnp.maximum(h + b_ref[...], 0.0)
        p2 = jnp.dot(segT_ref[...], h, preferred_element_type=jnp.float32)

        @pl.when(i == 0)
        def _():
            acc_ref[...] = p2

        @pl.when(i > 0)
        def _():
            acc_ref[...] += p2

        @pl.when(i == NB - 1)
        def _():
            preds = jnp.dot(ge0_ref[...], c0_ref[...],
                            preferred_element_type=jnp.float32)
            preds += jnp.dot(ge1_ref[...], c1_ref[...],
                             preferred_element_type=jnp.float32)
            preds += jnp.dot(acc_ref[...], c2_ref[...],
                             preferred_element_type=jnp.float32)
            preds_ref[...] = preds + cb_ref[...]

    return pl.pallas_call(
        body,
        grid=(NB,),
        in_specs=[
            pl.BlockSpec((RBLK, HID), lambda i: (i, 0)),
            pl.BlockSpec((G, RBLK), lambda i: (0, i)),
            pl.BlockSpec((HID, HID), lambda i: (0, 0)),
            pl.BlockSpec((1, HID), lambda i: (0, 0)),
            pl.BlockSpec((G, D), lambda i: (0, 0)),
            pl.BlockSpec((G, HID), lambda i: (0, 0)),
            pl.BlockSpec((D, OUT), lambda i: (0, 0)),
            pl.BlockSpec((HID, OUT), lambda i: (0, 0)),
            pl.BlockSpec((HID, OUT), lambda i: (0, 0)),
            pl.BlockSpec((1, OUT), lambda i: (0, 0)),
        ],
        out_specs=pl.BlockSpec((G, OUT), lambda i: (0, 0)),
        out_shape=jax.ShapeDtypeStruct((G, OUT), jnp.float32),
        scratch_shapes=[pltpu.VMEM((G, HID), jnp.float32)],
    )(agg2, segT, W1, b1, ge0, ge1, C0w, C1w, C2w, cb)


def kernel(x, neighbours, segment_ids, W0, b0, W1, b1,
           C0w, C0b, C1w, C1b, C2w, C2b):
    x_pad = jnp.zeros((NPAD, D), jnp.float32).at[:N].set(x)
    nbr_pad = jnp.zeros((NPAD, MAXD), jnp.int32).at[:N].set(neighbours)
    idx4 = nbr_pad.reshape(NW, NCH, CHUNK, MAXD).transpose(0, 1, 3, 2)
    seg_pad = jnp.full((NPAD,), G, jnp.int32).at[:N].set(segment_ids)
    segT = (seg_pad[None, :] == jnp.arange(G, dtype=jnp.int32)[:, None]
            ).astype(jnp.float32)

    agg1 = _sc_gather_sum(x_pad, idx4)
    H1, ge0, ge1 = _tc_layer1(agg1, x_pad, segT, W0, b0.reshape(1, HID))
    agg2 = _sc_gather_sum(H1, idx4)
    cb = (C0b + C1b + C2b).reshape(1, OUT)
    return _tc_layer2(agg2, segT, W1, b1.reshape(1, HID),
                      ge0, ge1, C0w, C1w, C2w, cb)


# speedup vs baseline: 2.1792x; 2.1792x over previous
import functools

import jax
import jax.numpy as jnp
from jax import lax
from jax.experimental import pallas as pl
from jax.experimental.pallas import tpu as pltpu
from jax.experimental.pallas import tpu_sc as plsc

N = 10000
D = 256
HID = 256
OUT = 64
MAXD = 17
G = 8

NC = 2
NS = 16
NW = NC * NS
PW = 320
NPAD = NW * PW
CHUNK = 8
NSUB = PW // CHUNK
LANES = 16
NCOL = D // LANES

RBLK = 1024
NB = NPAD // RBLK


def _sc_gather_sum(table, idx4):
    mesh = plsc.VectorSubcoreMesh(core_axis_name="c", subcore_axis_name="s")

    @functools.partial(
        pl.kernel,
        out_type=jax.ShapeDtypeStruct((NPAD, D), jnp.float32),
        mesh=mesh,
        scratch_types=[
            pltpu.VMEM((NSUB, 256), jnp.int32),
            pltpu.VMEM((2, MAXD, CHUNK, D), jnp.float32),
            pltpu.VMEM((2, CHUNK, D), jnp.float32),
            pltpu.SemaphoreType.DMA,
            pltpu.SemaphoreType.DMA,
            pltpu.SemaphoreType.DMA,
            pltpu.SemaphoreType.DMA,
        ],
    )
    def k(table_hbm, idx_hbm, out_hbm, idx_v, buf, obuf, g0, g1, o0, o1):
        wid = lax.axis_index("s") * NC + lax.axis_index("c")
        base = wid * PW
        gsem = (g0, g1)
        osem = (o0, o1)
        pltpu.sync_copy(idx_hbm.at[wid], idx_v)

        def fire(s, par, sem):
            for j in range(MAXD):
                pltpu.async_copy(table_hbm.at[idx_v.at[s, pl.ds(j * CHUNK, CHUNK)]],
                                 buf.at[par, j], sem)

        def drain_gathers(par, sem):
            for j in range(MAXD):
                pltpu.make_async_copy(table_hbm.at[pl.ds(0, CHUNK)],
                                      buf.at[par, j], sem).wait()

        def drain_writeout(par, sem):
            pltpu.make_async_copy(table_hbm.at[pl.ds(0, CHUNK)],
                                  obuf.at[par], sem).wait()

        def accumulate(par):
            def row(r, _):
                for c in range(NCOL):
                    sl = pl.ds(c * LANES, LANES)
                    v = buf[par, 0, r, sl]
                    for j in range(1, MAXD):
                        v = v + buf[par, j, r, sl]
                    obuf[par, r, sl] = v
                return _
            lax.fori_loop(0, CHUNK, row, 0, unroll=2)

        def phase(i, s, par):
            nxt = s + 1

            @pl.when(nxt < NSUB)
            def _():
                fire(nxt, 1 - par, gsem[1 - par])

            drain_gathers(par, gsem[par])

            @pl.when(i > 0)
            def _():
                drain_writeout(par, osem[par])

            accumulate(par)
            pltpu.async_copy(obuf.at[par], out_hbm.at[pl.ds(base + s * CHUNK, CHUNK)],
                             osem[par])

        fire(0, 0, g0)

        def body(i, _):
            phase(i, 2 * i, 0)
            phase(i, 2 * i + 1, 1)
            return _

        lax.fori_loop(0, NSUB // 2, body, 0)
        drain_writeout(0, o0)
        drain_writeout(1, o1)

    return k(table, idx4)


def _tc_layer1(agg1, x_pad, segT, W0, b0):
    def body(agg_ref, x_ref, segT_ref, w_ref, b_ref, h_ref, ge0_ref, ge1_ref):
        i = pl.program_id(0)
        h = jnp.dot(agg_ref[...], w_ref[...], preferred_element_type=jnp.float32)
        h = jnp.maximum(h + b_ref[...], 0.0)
        h_ref[...] = h
        s = segT_ref[...]
        p0 = jnp.dot(s, x_ref[...], preferred_element_type=jnp.float32)
        p1 = jnp.dot(s, h, preferred_element_type=jnp.float32)

        @pl.when(i == 0)
        def _():
            ge0_ref[...] = p0
            ge1_ref[...] = p1

        @pl.when(i > 0)
        def _():
            ge0_ref[...] += p0
            ge1_ref[...] += p1

    return pl.pallas_call(
        body,
        grid=(NB,),
        in_specs=[
            pl.BlockSpec((RBLK, D), lambda i: (i, 0)),
            pl.BlockSpec((RBLK, D), lambda i: (i, 0)),
            pl.BlockSpec((G, RBLK), lambda i: (0, i)),
            pl.BlockSpec((D, HID), lambda i: (0, 0)),
            pl.BlockSpec((1, HID), lambda i: (0, 0)),
        ],
        out_specs=[
            pl.BlockSpec((RBLK, HID), lambda i: (i, 0)),
            pl.BlockSpec((G, D), lambda i: (0, 0)),
            pl.BlockSpec((G, HID), lambda i: (0, 0)),
        ],
        out_shape=[
            jax.ShapeDtypeStruct((NPAD, HID), jnp.float32),
            jax.ShapeDtypeStruct((G, D), jnp.float32),
            jax.ShapeDtypeStruct((G, HID), jnp.float32),
        ],
    )(agg1, x_pad, segT, W0, b0)


def _tc_layer2(agg2, segT, W1, b1, ge0, ge1, C0w, C1w, C2w, cb):
    def body(agg_ref, segT_ref, w_ref, b_ref, ge0_ref, ge1_ref,
             c0_ref, c1_ref, c2_ref, cb_ref, preds_ref, acc_ref):
        i = pl.program_id(0)
        h = jnp.dot(agg_ref[...], w_ref[...], preferred_element_type=jnp.float32)
        h = jnp.maximum(h + b_ref[...], 0.0)
        p2 = jnp.dot(segT_ref[...], h, preferred_element_type=jnp.float32)

        @pl.when(i == 0)
        def _():
            acc_ref[...] = p2

        @pl.when(i > 0)
        def _():
            acc_ref[...] += p2

        @pl.when(i == NB - 1)
        def _():
            preds = jnp.dot(ge0_ref[...], c0_ref[...],
                            preferred_element_type=jnp.float32)
            preds += jnp.dot(ge1_ref[...], c1_ref[...],
                             preferred_element_type=jnp.float32)
            preds += jnp.dot(acc_ref[...], c2_ref[...],
                             preferred_element_type=jnp.float32)
            preds_ref[...] = preds + cb_ref[...]

    return pl.pallas_call(
        body,
        grid=(NB,),
        in_specs=[
            pl.BlockSpec((RBLK, HID), lambda i: (i, 0)),
            pl.BlockSpec((G, RBLK), lambda i: (0, i)),
            pl.BlockSpec((HID, HID), lambda i: (0, 0)),
            pl.BlockSpec((1, HID), lambda i: (0, 0)),
            pl.BlockSpec((G, D), lambda i: (0, 0)),
            pl.BlockSpec((G, HID), lambda i: (0, 0)),
            pl.BlockSpec((D, OUT), lambda i: (0, 0)),
            pl.BlockSpec((HID, OUT), lambda i: (0, 0)),
            pl.BlockSpec((HID, OUT), lambda i: (0, 0)),
            pl.BlockSpec((1, OUT), lambda i: (0, 0)),
        ],
        out_specs=pl.BlockSpec((G, OUT), lambda i: (0, 0)),
        out_shape=jax.ShapeDtypeStruct((G, OUT), jnp.float32),
        scratch_shapes=[pltpu.VMEM((G, HID), jnp.float32)],
    )(agg2, segT, W1, b1, ge0, ge1, C0w, C1w, C2w, cb)


def kernel(x, neighbours, segment_ids, W0, b0, W1, b1,
           C0w, C0b, C1w, C1b, C2w, C2b):
    x_pad = jnp.zeros((NPAD, D), jnp.float32).at[:N].set(x)
    nbr_pad = jnp.zeros((NPAD, MAXD), jnp.int32).at[:N].set(neighbours)
    idxp = nbr_pad.reshape(NW, NSUB, CHUNK, MAXD).transpose(0, 1, 3, 2)
    idx3 = jnp.zeros((NW, NSUB, 256), jnp.int32).at[:, :, :MAXD * CHUNK].set(
        idxp.reshape(NW, NSUB, MAXD * CHUNK))
    seg_pad = jnp.full((NPAD,), G, jnp.int32).at[:N].set(segment_ids)
    segT = (seg_pad[None, :] == jnp.arange(G, dtype=jnp.int32)[:, None]
            ).astype(jnp.float32)

    agg1 = _sc_gather_sum(x_pad, idx3)
    H1, ge0, ge1 = _tc_layer1(agg1, x_pad, segT, W0, b0.reshape(1, HID))
    agg2 = _sc_gather_sum(H1, idx3)
    cb = (C0b + C1b + C2b).reshape(1, OUT)
    return _tc_layer2(agg2, segT, W1, b1.reshape(1, HID),
                      ge0, ge1, C0w, C1w, C2w, cb)


# speedup vs baseline: 2.1828x; 1.0016x over previous
import functools

import jax
import jax.numpy as jnp
from jax import lax
from jax.experimental import pallas as pl
from jax.experimental.pallas import tpu as pltpu
from jax.experimental.pallas import tpu_sc as plsc

N = 10000
D = 256
HID = 256
OUT = 64
MAXD = 17
G = 8

NC = 2
NS = 16
NW = NC * NS
PW = 320
NPAD = NW * PW
CHUNK = 8
NSUB = PW // CHUNK
LANES = 16
NCOL = D // LANES

RBLK = 1024
NB = NPAD // RBLK


def _sc_gather_sum(table, idx4):
    mesh = plsc.VectorSubcoreMesh(core_axis_name="c", subcore_axis_name="s")

    @functools.partial(
        pl.kernel,
        out_type=jax.ShapeDtypeStruct((NPAD, D), jnp.float32),
        mesh=mesh,
        scratch_types=[
            pltpu.VMEM((NSUB, 256), jnp.int32),
            pltpu.VMEM((2, MAXD * CHUNK, D), jnp.float32),
            pltpu.VMEM((2, CHUNK, D), jnp.float32),
            pltpu.SemaphoreType.DMA,
            pltpu.SemaphoreType.DMA,
            pltpu.SemaphoreType.DMA,
            pltpu.SemaphoreType.DMA,
        ],
    )
    def k(table_hbm, idx_hbm, out_hbm, idx_v, buf, obuf, g0, g1, o0, o1):
        wid = lax.axis_index("s") * NC + lax.axis_index("c")
        base = wid * PW
        gsem = (g0, g1)
        osem = (o0, o1)
        pltpu.sync_copy(idx_hbm.at[wid], idx_v)

        SPLITS = ((0, 128), (128, MAXD * CHUNK - 128))

        def fire(s, par, sem):
            for lo, n in SPLITS:
                pltpu.async_copy(table_hbm.at[idx_v.at[s, pl.ds(lo, n)]],
                                 buf.at[par, pl.ds(lo, n)], sem)

        def drain_gathers(par, sem):
            for lo, n in SPLITS:
                pltpu.make_async_copy(table_hbm.at[pl.ds(0, n)],
                                      buf.at[par, pl.ds(lo, n)], sem).wait()

        def drain_writeout(par, sem):
            pltpu.make_async_copy(table_hbm.at[pl.ds(0, CHUNK)],
                                  obuf.at[par], sem).wait()

        def accumulate(par):
            def row(r, _):
                for c in range(NCOL):
                    sl = pl.ds(c * LANES, LANES)
                    v = buf[par, r, sl]
                    for j in range(1, MAXD):
                        v = v + buf[par, j * CHUNK + r, sl]
                    obuf[par, r, sl] = v
                return _
            lax.fori_loop(0, CHUNK, row, 0, unroll=2)

        def phase(i, s, par):
            nxt = s + 1

            @pl.when(nxt < NSUB)
            def _():
                fire(nxt, 1 - par, gsem[1 - par])

            drain_gathers(par, gsem[par])

            @pl.when(i > 0)
            def _():
                drain_writeout(par, osem[par])

            accumulate(par)
            pltpu.async_copy(obuf.at[par], out_hbm.at[pl.ds(base + s * CHUNK, CHUNK)],
                             osem[par])

        fire(0, 0, g0)

        def body(i, _):
            phase(i, 2 * i, 0)
            phase(i, 2 * i + 1, 1)
            return _

        lax.fori_loop(0, NSUB // 2, body, 0)
        drain_writeout(0, o0)
        drain_writeout(1, o1)

    return k(table, idx4)


def _tc_layer1(agg1, x_pad, segT, W0, b0):
    def body(agg_ref, x_ref, segT_ref, w_ref, b_ref, h_ref, ge0_ref, ge1_ref):
        i = pl.program_id(0)
        h = jnp.dot(agg_ref[...], w_ref[...], preferred_element_type=jnp.float32)
        h = jnp.maximum(h + b_ref[...], 0.0)
        h_ref[...] = h
        s = segT_ref[...]
        p0 = jnp.dot(s, x_ref[...], preferred_element_type=jnp.float32)
        p1 = jnp.dot(s, h, preferred_element_type=jnp.float32)

        @pl.when(i == 0)
        def _():
            ge0_ref[...] = p0
            ge1_ref[...] = p1

        @pl.when(i > 0)
        def _():
            ge0_ref[...] += p0
            ge1_ref[...] += p1

    return pl.pallas_call(
        body,
        grid=(NB,),
        in_specs=[
            pl.BlockSpec((RBLK, D), lambda i: (i, 0)),
            pl.BlockSpec((RBLK, D), lambda i: (i, 0)),
            pl.BlockSpec((G, RBLK), lambda i: (0, i)),
            pl.BlockSpec((D, HID), lambda i: (0, 0)),
            pl.BlockSpec((1, HID), lambda i: (0, 0)),
        ],
        out_specs=[
            pl.BlockSpec((RBLK, HID), lambda i: (i, 0)),
            pl.BlockSpec((G, D), lambda i: (0, 0)),
            pl.BlockSpec((G, HID), lambda i: (0, 0)),
        ],
        out_shape=[
            jax.ShapeDtypeStruct((NPAD, HID), jnp.float32),
            jax.ShapeDtypeStruct((G, D), jnp.float32),
            jax.ShapeDtypeStruct((G, HID), jnp.float32),
        ],
    )(agg1, x_pad, segT, W0, b0)


def _tc_layer2(agg2, segT, W1, b1, ge0, ge1, C0w, C1w, C2w, cb):
    def body(agg_ref, segT_ref, w_ref, b_ref, ge0_ref, ge1_ref,
             c0_ref, c1_ref, c2_ref, cb_ref, preds_ref, acc_ref):
        i = pl.program_id(0)
        h = jnp.dot(agg_ref[...], w_ref[...], preferred_element_type=jnp.float32)
        h = jnp.maximum(h + b_ref[...], 0.0)
        p2 = jnp.dot(segT_ref[...], h, preferred_element_type=jnp.float32)

        @pl.when(i == 0)
        def _():
            acc_ref[...] = p2

        @pl.when(i > 0)
        def _():
            acc_ref[...] += p2

        @pl.when(i == NB - 1)
        def _():
            preds = jnp.dot(ge0_ref[...], c0_ref[...],
                            preferred_element_type=jnp.float32)
            preds += jnp.dot(ge1_ref[...], c1_ref[...],
                             preferred_element_type=jnp.float32)
            preds += jnp.dot(acc_ref[...], c2_ref[...],
                             preferred_element_type=jnp.float32)
            preds_ref[...] = preds + cb_ref[...]

    return pl.pallas_call(
        body,
        grid=(NB,),
        in_specs=[
            pl.BlockSpec((RBLK, HID), lambda i: (i, 0)),
            pl.BlockSpec((G, RBLK), lambda i: (0, i)),
            pl.BlockSpec((HID, HID), lambda i: (0, 0)),
            pl.BlockSpec((1, HID), lambda i: (0, 0)),
            pl.BlockSpec((G, D), lambda i: (0, 0)),
            pl.BlockSpec((G, HID), lambda i: (0, 0)),
            pl.BlockSpec((D, OUT), lambda i: (0, 0)),
            pl.BlockSpec((HID, OUT), lambda i: (0, 0)),
            pl.BlockSpec((HID, OUT), lambda i: (0, 0)),
            pl.BlockSpec((1, OUT), lambda i: (0, 0)),
        ],
        out_specs=pl.BlockSpec((G, OUT), lambda i: (0, 0)),
        out_shape=jax.ShapeDtypeStruct((G, OUT), jnp.float32),
        scratch_shapes=[pltpu.VMEM((G, HID), jnp.float32)],
    )(agg2, segT, W1, b1, ge0, ge1, C0w, C1w, C2w, cb)


def kernel(x, neighbours, segment_ids, W0, b0, W1, b1,
           C0w, C0b, C1w, C1b, C2w, C2b):
    x_pad = jnp.zeros((NPAD, D), jnp.float32).at[:N].set(x)
    nbr_pad = jnp.zeros((NPAD, MAXD), jnp.int32).at[:N].set(neighbours)
    idxp = nbr_pad.reshape(NW, NSUB, CHUNK, MAXD).transpose(0, 1, 3, 2)
    idx3 = jnp.zeros((NW, NSUB, 256), jnp.int32).at[:, :, :MAXD * CHUNK].set(
        idxp.reshape(NW, NSUB, MAXD * CHUNK))
    seg_pad = jnp.full((NPAD,), G, jnp.int32).at[:N].set(segment_ids)
    segT = (seg_pad[None, :] == jnp.arange(G, dtype=jnp.int32)[:, None]
            ).astype(jnp.float32)

    agg1 = _sc_gather_sum(x_pad, idx3)
    H1, ge0, ge1 = _tc_layer1(agg1, x_pad, segT, W0, b0.reshape(1, HID))
    agg2 = _sc_gather_sum(H1, idx3)
    cb = (C0b + C1b + C2b).reshape(1, OUT)
    return _tc_layer2(agg2, segT, W1, b1.reshape(1, HID),
                      ge0, ge1, C0w, C1w, C2w, cb)


# speedup vs baseline: 2.1876x; 1.0022x over previous
import functools

import jax
import jax.numpy as jnp
from jax import lax
from jax.experimental import pallas as pl
from jax.experimental.pallas import tpu as pltpu
from jax.experimental.pallas import tpu_sc as plsc

N = 10000
D = 256
HID = 256
OUT = 64
MAXD = 17
G = 8

NC = 2
NS = 16
NW = NC * NS
PW = 320
NPAD = NW * PW
CHUNK = 8
NSUB = PW // CHUNK
LANES = 16
NCOL = D // LANES

RBLK = 1024
NB = NPAD // RBLK


def _sc_gather_sum(table, idx4):
    mesh = plsc.VectorSubcoreMesh(core_axis_name="c", subcore_axis_name="s")

    @functools.partial(
        pl.kernel,
        out_type=jax.ShapeDtypeStruct((NPAD, D), jnp.float32),
        mesh=mesh,
        scratch_types=[
            pltpu.VMEM((NSUB, 256), jnp.int32),
            pltpu.VMEM((2, MAXD * CHUNK, D), jnp.float32),
            pltpu.VMEM((2, CHUNK, D), jnp.float32),
            pltpu.SemaphoreType.DMA,
            pltpu.SemaphoreType.DMA,
            pltpu.SemaphoreType.DMA,
            pltpu.SemaphoreType.DMA,
        ],
    )
    def k(table_hbm, idx_hbm, out_hbm, idx_v, buf, obuf, g0, g1, o0, o1):
        wid = lax.axis_index("s") * NC + lax.axis_index("c")
        base = wid * PW
        gsem = (g0, g1)
        osem = (o0, o1)
        pltpu.sync_copy(idx_hbm.at[wid], idx_v)

        SPLITS = ((0, 128), (128, MAXD * CHUNK - 128))

        def fire(s, par, sem):
            for lo, n in SPLITS:
                pltpu.async_copy(table_hbm.at[idx_v.at[s, pl.ds(lo, n)]],
                                 buf.at[par, pl.ds(lo, n)], sem)

        def drain_gathers(par, sem):
            for lo, n in SPLITS:
                pltpu.make_async_copy(table_hbm.at[pl.ds(0, n)],
                                      buf.at[par, pl.ds(lo, n)], sem).wait()

        def drain_writeout(par, sem):
            pltpu.make_async_copy(table_hbm.at[pl.ds(0, CHUNK)],
                                  obuf.at[par], sem).wait()

        def accumulate(par):
            def row(r, _):
                for c in range(NCOL):
                    sl = pl.ds(c * LANES, LANES)
                    v = buf[par, r, sl]
                    for j in range(1, MAXD):
                        v = v + buf[par, j * CHUNK + r, sl]
                    obuf[par, r, sl] = v
                return _
            lax.fori_loop(0, CHUNK, row, 0, unroll=2)

        def phase(i, s, par):
            nxt = s + 1

            @pl.when(nxt < NSUB)
            def _():
                fire(nxt, 1 - par, gsem[1 - par])

            with jax.named_scope("drain_g"):
                drain_gathers(par, gsem[par])

            @pl.when(i > 0)
            def _():
                drain_writeout(par, osem[par])

            with jax.named_scope("acc"):
                accumulate(par)
            pltpu.async_copy(obuf.at[par], out_hbm.at[pl.ds(base + s * CHUNK, CHUNK)],
                             osem[par])

        fire(0, 0, g0)

        def body(i, _):
            phase(i, 2 * i, 0)
            phase(i, 2 * i + 1, 1)
            return _

        lax.fori_loop(0, NSUB // 2, body, 0)
        drain_writeout(0, o0)
        drain_writeout(1, o1)

    return k(table, idx4)


def _tc_layer1(agg1, x_pad, segT, W0, b0):
    def body(agg_ref, x_ref, segT_ref, w_ref, b_ref, h_ref, ge0_ref, ge1_ref):
        i = pl.program_id(0)
        h = jnp.dot(agg_ref[...], w_ref[...], preferred_element_type=jnp.float32)
        h = jnp.maximum(h + b_ref[...], 0.0)
        h_ref[...] = h
        s = segT_ref[...]
        p0 = jnp.dot(s, x_ref[...], preferred_element_type=jnp.float32)
        p1 = jnp.dot(s, h, preferred_element_type=jnp.float32)

        @pl.when(i == 0)
        def _():
            ge0_ref[...] = p0
            ge1_ref[...] = p1

        @pl.when(i > 0)
        def _():
            ge0_ref[...] += p0
            ge1_ref[...] += p1

    return pl.pallas_call(
        body,
        grid=(NB,),
        in_specs=[
            pl.BlockSpec((RBLK, D), lambda i: (i, 0)),
            pl.BlockSpec((RBLK, D), lambda i: (i, 0)),
            pl.BlockSpec((G, RBLK), lambda i: (0, i)),
            pl.BlockSpec((D, HID), lambda i: (0, 0)),
            pl.BlockSpec((1, HID), lambda i: (0, 0)),
        ],
        out_specs=[
            pl.BlockSpec((RBLK, HID), lambda i: (i, 0)),
            pl.BlockSpec((G, D), lambda i: (0, 0)),
            pl.BlockSpec((G, HID), lambda i: (0, 0)),
        ],
        out_shape=[
            jax.ShapeDtypeStruct((NPAD, HID), jnp.float32),
            jax.ShapeDtypeStruct((G, D), jnp.float32),
            jax.ShapeDtypeStruct((G, HID), jnp.float32),
        ],
    )(agg1, x_pad, segT, W0, b0)


def _tc_layer2(agg2, segT, W1, b1, ge0, ge1, C0w, C1w, C2w, cb):
    def body(agg_ref, segT_ref, w_ref, b_ref, ge0_ref, ge1_ref,
             c0_ref, c1_ref, c2_ref, cb_ref, preds_ref, acc_ref):
        i = pl.program_id(0)
        h = jnp.dot(agg_ref[...], w_ref[...], preferred_element_type=jnp.float32)
        h = jnp.maximum(h + b_ref[...], 0.0)
        p2 = jnp.dot(segT_ref[...], h, preferred_element_type=jnp.float32)

        @pl.when(i == 0)
        def _():
            acc_ref[...] = p2

        @pl.when(i > 0)
        def _():
            acc_ref[...] += p2

        @pl.when(i == NB - 1)
        def _():
            preds = jnp.dot(ge0_ref[...], c0_ref[...],
                            preferred_element_type=jnp.float32)
            preds += jnp.dot(ge1_ref[...], c1_ref[...],
                             preferred_element_type=jnp.float32)
            preds += jnp.dot(acc_ref[...], c2_ref[...],
                             preferred_element_type=jnp.float32)
            preds_ref[...] = preds + cb_ref[...]

    return pl.pallas_call(
        body,
        grid=(NB,),
        in_specs=[
            pl.BlockSpec((RBLK, HID), lambda i: (i, 0)),
            pl.BlockSpec((G, RBLK), lambda i: (0, i)),
            pl.BlockSpec((HID, HID), lambda i: (0, 0)),
            pl.BlockSpec((1, HID), lambda i: (0, 0)),
            pl.BlockSpec((G, D), lambda i: (0, 0)),
            pl.BlockSpec((G, HID), lambda i: (0, 0)),
            pl.BlockSpec((D, OUT), lambda i: (0, 0)),
            pl.BlockSpec((HID, OUT), lambda i: (0, 0)),
            pl.BlockSpec((HID, OUT), lambda i: (0, 0)),
            pl.BlockSpec((1, OUT), lambda i: (0, 0)),
        ],
        out_specs=pl.BlockSpec((G, OUT), lambda i: (0, 0)),
        out_shape=jax.ShapeDtypeStruct((G, OUT), jnp.float32),
        scratch_shapes=[pltpu.VMEM((G, HID), jnp.float32)],
    )(agg2, segT, W1, b1, ge0, ge1, C0w, C1w, C2w, cb)


def kernel(x, neighbours, segment_ids, W0, b0, W1, b1,
           C0w, C0b, C1w, C1b, C2w, C2b):
    x_pad = jnp.zeros((NPAD, D), jnp.float32).at[:N].set(x)
    nbr_pad = jnp.zeros((NPAD, MAXD), jnp.int32).at[:N].set(neighbours)
    idxp = nbr_pad.reshape(NW, NSUB, CHUNK, MAXD).transpose(0, 1, 3, 2)
    idx3 = jnp.zeros((NW, NSUB, 256), jnp.int32).at[:, :, :MAXD * CHUNK].set(
        idxp.reshape(NW, NSUB, MAXD * CHUNK))
    seg_pad = jnp.full((NPAD,), G, jnp.int32).at[:N].set(segment_ids)
    segT = (seg_pad[None, :] == jnp.arange(G, dtype=jnp.int32)[:, None]
            ).astype(jnp.float32)

    agg1 = _sc_gather_sum(x_pad, idx3)
    H1, ge0, ge1 = _tc_layer1(agg1, x_pad, segT, W0, b0.reshape(1, HID))
    agg2 = _sc_gather_sum(H1, idx3)
    cb = (C0b + C1b + C2b).reshape(1, OUT)
    return _tc_layer2(agg2, segT, W1, b1.reshape(1, HID),
                      ge0, ge1, C0w, C1w, C2w, cb)


# speedup vs baseline: 4.8869x; 2.2340x over previous
import functools

import jax
import jax.numpy as jnp
from jax import lax
from jax.experimental import pallas as pl
from jax.experimental.pallas import tpu as pltpu
from jax.experimental.pallas import tpu_sc as plsc

N = 10000
D = 256
HID = 256
OUT = 64
MAXD = 17
G = 8

NC = 2
NS = 16
NW = NC * NS
PW = 320
NPAD = NW * PW
CHUNK = 8
NSUB = PW // CHUNK
LANES = 16
NCOL = D // LANES

RBLK = 1024
NB = NPAD // RBLK


def _sc_gather_sum(table, idx4):
    mesh = plsc.VectorSubcoreMesh(core_axis_name="c", subcore_axis_name="s")

    @functools.partial(
        pl.kernel,
        out_type=jax.ShapeDtypeStruct((NPAD, D), jnp.float32),
        mesh=mesh,
        scratch_types=[
            pltpu.VMEM((NSUB, 256), jnp.int32),
            pltpu.VMEM((2, MAXD * CHUNK, D), jnp.float32),
            pltpu.VMEM((2, CHUNK, D), jnp.float32),
            pltpu.SemaphoreType.DMA,
            pltpu.SemaphoreType.DMA,
            pltpu.SemaphoreType.DMA,
            pltpu.SemaphoreType.DMA,
        ],
    )
    def k(table_hbm, idx_hbm, out_hbm, idx_v, buf, obuf, g0, g1, o0, o1):
        wid = lax.axis_index("s") * NC + lax.axis_index("c")
        base = wid * PW
        gsem = (g0, g1)
        osem = (o0, o1)
        pltpu.sync_copy(idx_hbm.at[wid], idx_v)

        SPLITS = ((0, 128), (128, MAXD * CHUNK - 128))

        def fire(s, par, sem):
            for lo, n in SPLITS:
                pltpu.async_copy(table_hbm.at[idx_v.at[s, pl.ds(lo, n)]],
                                 buf.at[par, pl.ds(lo, n)], sem)

        def drain_gathers(par, sem):
            for lo, n in SPLITS:
                pltpu.make_async_copy(table_hbm.at[pl.ds(0, n)],
                                      buf.at[par, pl.ds(lo, n)], sem).wait()

        def drain_writeout(par, sem):
            pltpu.make_async_copy(table_hbm.at[pl.ds(0, CHUNK)],
                                  obuf.at[par], sem).wait()

        def accumulate(par):
            def row(r, _):
                for c in range(NCOL):
                    sl = pl.ds(c * LANES, LANES)
                    v = buf[par, r, sl]
                    for j in range(1, MAXD):
                        v = v + buf[par, j * CHUNK + r, sl]
                    obuf[par, r, sl] = v
                return _
            lax.fori_loop(0, CHUNK, row, 0, unroll=2)

        def phase(i, s, par):
            nxt = s + 1

            @pl.when(nxt < NSUB)
            def _():
                fire(nxt, 1 - par, gsem[1 - par])

            with jax.named_scope("drain_g"):
                drain_gathers(par, gsem[par])

            @pl.when(i > 0)
            def _():
                drain_writeout(par, osem[par])

            with jax.named_scope("acc"):
                accumulate(par)
            pltpu.async_copy(obuf.at[par], out_hbm.at[pl.ds(base + s * CHUNK, CHUNK)],
                             osem[par])

        fire(0, 0, g0)

        def body(i, _):
            phase(i, 2 * i, 0)
            phase(i, 2 * i + 1, 1)
            return _

        lax.fori_loop(0, NSUB // 2, body, 0)
        drain_writeout(0, o0)
        drain_writeout(1, o1)

    return k(table, idx4)


def _tc_layer1(agg1, x_pad, segT, W0, b0):
    def body(agg_ref, x_ref, segT_ref, w_ref, b_ref, h_ref, ge0_ref, ge1_ref):
        i = pl.program_id(0)
        h = jnp.dot(agg_ref[...], w_ref[...], preferred_element_type=jnp.float32)
        h = jnp.maximum(h + b_ref[...], 0.0)
        h_ref[...] = h
        s = segT_ref[...]
        p0 = jnp.dot(s, x_ref[...], preferred_element_type=jnp.float32)
        p1 = jnp.dot(s, h, preferred_element_type=jnp.float32)

        @pl.when(i == 0)
        def _():
            ge0_ref[...] = p0
            ge1_ref[...] = p1

        @pl.when(i > 0)
        def _():
            ge0_ref[...] += p0
            ge1_ref[...] += p1

    return pl.pallas_call(
        body,
        grid=(NB,),
        in_specs=[
            pl.BlockSpec((RBLK, D), lambda i: (i, 0)),
            pl.BlockSpec((RBLK, D), lambda i: (i, 0)),
            pl.BlockSpec((G, RBLK), lambda i: (0, i)),
            pl.BlockSpec((D, HID), lambda i: (0, 0)),
            pl.BlockSpec((1, HID), lambda i: (0, 0)),
        ],
        out_specs=[
            pl.BlockSpec((RBLK, HID), lambda i: (i, 0)),
            pl.BlockSpec((G, D), lambda i: (0, 0)),
            pl.BlockSpec((G, HID), lambda i: (0, 0)),
        ],
        out_shape=[
            jax.ShapeDtypeStruct((NPAD, HID), jnp.float32),
            jax.ShapeDtypeStruct((G, D), jnp.float32),
            jax.ShapeDtypeStruct((G, HID), jnp.float32),
        ],
    )(agg1, x_pad, segT, W0, b0)


def _tc_layer2(agg2, segT, W1, b1, ge0, ge1, C0w, C1w, C2w, cb):
    def body(agg_ref, segT_ref, w_ref, b_ref, ge0_ref, ge1_ref,
             c0_ref, c1_ref, c2_ref, cb_ref, preds_ref, acc_ref):
        i = pl.program_id(0)
        h = jnp.dot(agg_ref[...], w_ref[...], preferred_element_type=jnp.float32)
        h = jnp.maximum(h + b_ref[...], 0.0)
        p2 = jnp.dot(segT_ref[...], h, preferred_element_type=jnp.float32)

        @pl.when(i == 0)
        def _():
            acc_ref[...] = p2

        @pl.when(i > 0)
        def _():
            acc_ref[...] += p2

        @pl.when(i == NB - 1)
        def _():
            preds = jnp.dot(ge0_ref[...], c0_ref[...],
                            preferred_element_type=jnp.float32)
            preds += jnp.dot(ge1_ref[...], c1_ref[...],
                             preferred_element_type=jnp.float32)
            preds += jnp.dot(acc_ref[...], c2_ref[...],
                             preferred_element_type=jnp.float32)
            preds_ref[...] = preds + cb_ref[...]

    return pl.pallas_call(
        body,
        grid=(NB,),
        in_specs=[
            pl.BlockSpec((RBLK, HID), lambda i: (i, 0)),
            pl.BlockSpec((G, RBLK), lambda i: (0, i)),
            pl.BlockSpec((HID, HID), lambda i: (0, 0)),
            pl.BlockSpec((1, HID), lambda i: (0, 0)),
            pl.BlockSpec((G, D), lambda i: (0, 0)),
            pl.BlockSpec((G, HID), lambda i: (0, 0)),
            pl.BlockSpec((D, OUT), lambda i: (0, 0)),
            pl.BlockSpec((HID, OUT), lambda i: (0, 0)),
            pl.BlockSpec((HID, OUT), lambda i: (0, 0)),
            pl.BlockSpec((1, OUT), lambda i: (0, 0)),
        ],
        out_specs=pl.BlockSpec((G, OUT), lambda i: (0, 0)),
        out_shape=jax.ShapeDtypeStruct((G, OUT), jnp.float32),
        scratch_shapes=[pltpu.VMEM((G, HID), jnp.float32)],
    )(agg2, segT, W1, b1, ge0, ge1, C0w, C1w, C2w, cb)


def kernel(x, neighbours, segment_ids, W0, b0, W1, b1,
           C0w, C0b, C1w, C1b, C2w, C2b):
    x_pad = jnp.zeros((NPAD, D), jnp.float32).at[:N].set(x)
    spread = (jnp.arange((NPAD - N) * MAXD, dtype=jnp.int32) * 37) % N
    nbr_pad = jnp.concatenate(
        [neighbours, spread.reshape(NPAD - N, MAXD)], axis=0)
    idxp = nbr_pad.reshape(NW, NSUB, CHUNK, MAXD).transpose(0, 1, 3, 2)
    idx3 = jnp.zeros((NW, NSUB, 256), jnp.int32).at[:, :, :MAXD * CHUNK].set(
        idxp.reshape(NW, NSUB, MAXD * CHUNK))
    seg_pad = jnp.full((NPAD,), G, jnp.int32).at[:N].set(segment_ids)
    segT = (seg_pad[None, :] == jnp.arange(G, dtype=jnp.int32)[:, None]
            ).astype(jnp.float32)

    agg1 = _sc_gather_sum(x_pad, idx3)
    H1, ge0, ge1 = _tc_layer1(agg1, x_pad, segT, W0, b0.reshape(1, HID))
    agg2 = _sc_gather_sum(H1, idx3)
    cb = (C0b + C1b + C2b).reshape(1, OUT)
    return _tc_layer2(agg2, segT, W1, b1.reshape(1, HID),
                      ge0, ge1, C0w, C1w, C2w, cb)
